# baseline (speedup 1.0000x reference)
import functools

import jax
import jax.numpy as jnp
from jax import lax
from jax.experimental import pallas as pl
from jax.experimental.pallas import tpu as pltpu
from jax.experimental.pallas import tpu_sc as plsc

NUM_LANGUAGES = 4
VOCAB = 100000
DIM = 512
BATCH = 4096
SEQ = 50

N_ROWS = BATCH * SEQ
SPLIT = 2
HDIM = DIM // SPLIT
N_IDX = N_ROWS * SPLIT
WINDOW = 128


def _gather_rows(table_flat, indices):
    mesh = plsc.VectorSubcoreMesh(core_axis_name="c", subcore_axis_name="s")

    @functools.partial(
        pl.kernel,
        out_type=jax.ShapeDtypeStruct((N_IDX, HDIM), jnp.float32),
        mesh=mesh,
    )
    def k(table_hbm, idx_hbm, out_hbm):
        def body(idx_vmem, out_vmem):
            pltpu.sync_copy(table_hbm.at[idx_vmem.at[0]], out_vmem)

        pltpu.emit_pipeline(
            body,
            grid=(N_IDX // WINDOW,),
            in_specs=[pl.BlockSpec((1, WINDOW), index_map=lambda i: (0, i))],
            out_specs=[pl.BlockSpec((WINDOW, HDIM), index_map=lambda i: (i, 0))],
            core_axis_name=("c", "s"),
            dimension_semantics=(pltpu.PARALLEL,),
        )(idx_hbm, out_hbm)

    return k(table_flat, indices)


def kernel(tables, tokens, language_id):
    table_flat = tables.reshape(NUM_LANGUAGES * VOCAB * SPLIT, HDIM)
    row_idx = (tokens.astype(jnp.int32) + jnp.int32(language_id) * VOCAB).reshape(
        N_ROWS
    )
    half_idx = (SPLIT * row_idx[:, None] + jnp.arange(SPLIT, dtype=jnp.int32)).reshape(
        1, N_IDX
    )
    rows = _gather_rows(table_flat, half_idx)
    shared_embedding = rows.reshape(BATCH, SEQ, DIM)
    language_prediction = jax.random.normal(
        jax.random.key(42), (1, 10), dtype=jnp.float32
    )
    return (shared_embedding, language_prediction)


# speedup vs baseline: 2.1192x; 2.1192x over previous
import functools

import jax
import jax.numpy as jnp
from jax.experimental import pallas as pl
from jax.experimental.pallas import tpu as pltpu
from jax.experimental.pallas import tpu_sc as plsc

NUM_LANGUAGES = 4
VOCAB = 100000
DIM = 512
BATCH = 4096
SEQ = 50


WINDOW = 128
HALF = WINDOW // 2
N_WIN = BATCH * SEQ // WINDOW


def _gather_rows(table_flat, idxp):
    mesh = plsc.VectorSubcoreMesh(core_axis_name="c", subcore_axis_name="s")

    @functools.partial(
        pl.kernel,
        out_type=jax.ShapeDtypeStruct((BATCH * SEQ, DIM), jnp.float32),
        mesh=mesh,
    )
    def k(table_hbm, idx_hbm, out_hbm):
        def body(idx_vmem, out_vmem):
            j = pl.program_id(1)
            idx_s = idx_vmem.at[0, 0, pl.ds(j * HALF, HALF)]
            pltpu.sync_copy(table_hbm.at[idx_s], out_vmem)

        pltpu.emit_pipeline(
            body,
            grid=(N_WIN, 2),
            in_specs=[pl.BlockSpec((1, 1, WINDOW), index_map=lambda i, j: (i, 0, 0))],
            out_specs=[pl.BlockSpec((HALF, DIM), index_map=lambda i, j: (2 * i + j, 0))],
            core_axis_name=("c", "s"),
            dimension_semantics=(pltpu.PARALLEL, pltpu.ARBITRARY),
        )(idx_hbm, out_hbm)

    return k(table_flat, idxp)


def kernel(tables, tokens, language_id):
    table_flat = tables.reshape(NUM_LANGUAGES * VOCAB, DIM)
    tok_off = tokens.astype(jnp.int32) + jnp.int32(language_id) * VOCAB
    idxp = tok_off.reshape(N_WIN, 1, WINDOW)
    shared_embedding = _gather_rows(table_flat, idxp).reshape(BATCH, SEQ, DIM)
    language_prediction = jax.random.normal(
        jax.random.key(42), (1, 10), dtype=jnp.float32
    )
    return (shared_embedding, language_prediction)


# speedup vs baseline: 6.6100x; 3.1191x over previous
import functools

import jax
import jax.numpy as jnp
from jax.experimental import pallas as pl
from jax.experimental.pallas import tpu as pltpu
from jax.experimental.pallas import tpu_sc as plsc

NUM_LANGUAGES = 4
VOCAB = 100000
DIM = 512
BATCH = 4096
SEQ = 50


WINDOW = 128
HALF = WINDOW // 2
N_WIN = BATCH * SEQ // WINDOW


def _gather_rows(table_flat, idxp):
    mesh = plsc.VectorSubcoreMesh(core_axis_name="c", subcore_axis_name="s")

    @functools.partial(
        pl.kernel,
        out_type=jax.ShapeDtypeStruct((BATCH * SEQ, DIM), jnp.float32),
        mesh=mesh,
    )
    def k(table_hbm, idx_hbm, out_hbm):
        def body(idx_vmem, out_vmem):
            j = pl.program_id(1)
            idx_s = idx_vmem.at[0, 0, pl.ds(j * HALF, HALF)]
            pltpu.sync_copy(table_hbm.at[idx_s], out_vmem)

        pltpu.emit_pipeline(
            body,
            grid=(N_WIN, 2),
            in_specs=[pl.BlockSpec((1, 1, WINDOW), index_map=lambda i, j: (i, 0, 0))],
            out_specs=[pl.BlockSpec((HALF, DIM), index_map=lambda i, j: (2 * i + j, 0))],
            core_axis_name=("c", "s"),
            dimension_semantics=(pltpu.PARALLEL, pltpu.ARBITRARY),
        )(idx_hbm, out_hbm)

    return k(table_flat, idxp)


def kernel(tables, tokens, language_id):
    table_flat = tables.reshape(NUM_LANGUAGES * VOCAB, DIM)
    tok_off = tokens.astype(jnp.int32) + jnp.int32(language_id) * VOCAB
    idxp = tok_off.T.reshape(N_WIN, 1, WINDOW)
    rows = _gather_rows(table_flat, idxp)
    shared_embedding = rows.reshape(SEQ, BATCH, DIM).transpose(1, 0, 2)
    language_prediction = jax.random.normal(
        jax.random.key(42), (1, 10), dtype=jnp.float32
    )
    return (shared_embedding, language_prediction)
